# Initial kernel scaffold; baseline (speedup 1.0000x reference)
import jax
import jax.numpy as jnp
from jax.experimental import pallas as pl


def kernel(x, expander_edge_index, expander_edge_attr, W_Q, W_K, W_E, W_V, ln1_g, ln1_b, W1, b1, W2, b2, ln2_g, ln2_b):
    raise NotImplementedError("write your pallas kernel here")


# speedup vs baseline: 37.3054x; 37.3054x over previous
import functools

import jax
import jax.numpy as jnp
import numpy as np
from jax import lax
from jax.experimental import pallas as pl
from jax.experimental.pallas import tpu as pltpu
from jax.experimental.pallas import tpu_sc as plsc

N = 10000
E = 320000
D = 128
H = 8
DH = 16
DE = 16
DW = D + DH

NC = 2
NS = 16
NW = NC * NS
EW = E // NW
CH = 80
HFA = 48
HFB = CH - HFA
NCHUNK = EW // CH
RPT = 624

_f32 = jnp.float32


def _proj_body(x_ref, w_ref, q_ref, k_ref, v_ref):
    out = jnp.dot(x_ref[...], w_ref[...], preferred_element_type=_f32)
    q_ref[...] = out[:, :D]
    k_ref[...] = out[:, D:2 * D]
    v_ref[...] = out[:, 2 * D:]


def _project(x, w_all):
    bn = 5000
    return pl.pallas_call(
        _proj_body,
        grid=(N // bn,),
        in_specs=[
            pl.BlockSpec((bn, D), lambda i: (i, 0)),
            pl.BlockSpec((D, 3 * D), lambda i: (0, 0)),
        ],
        out_specs=[pl.BlockSpec((bn, D), lambda i: (i, 0))] * 3,
        out_shape=[jax.ShapeDtypeStruct((N, D), _f32)] * 3,
    )(x, w_all)


_MESH = plsc.VectorSubcoreMesh(core_axis_name="c", subcore_axis_name="s")


@functools.partial(
    pl.kernel,
    out_type=jax.ShapeDtypeStruct((2, E, D), _f32),
    mesh=_MESH,
    scratch_types=[
        pltpu.VMEM((2, 256), jnp.int32),
        pltpu.VMEM((2, CH, D), _f32),
        pltpu.VMEM((CH, D), _f32),
        pltpu.VMEM_SHARED((N, D), _f32),
        pltpu.SemaphoreType.DMA,
        pltpu.SemaphoreType.DMA,
        pltpu.SemaphoreType.DMA,
    ],
)
def _gather_kq(k_hbm, q_hbm, epk_hbm, kq_out,
               eidx2, rows2, bnc, tbl_sh, semi, semg, semw):
    wid = lax.axis_index("s") * NC + lax.axis_index("c")
    sid = lax.axis_index("s")
    base = wid * EW
    cbase = wid * NCHUNK
    r0 = sid * RPT

    def stage_tbl(src_hbm):
        def cp(j, c):
            pltpu.sync_copy(src_hbm.at[pl.ds(r0 + j * CH, CH)], bnc)
            pltpu.sync_copy(bnc, tbl_sh.at[pl.ds(r0 + j * CH, CH)])
            return c

        lax.fori_loop(0, 7, cp, 0)

        @pl.when(sid < NS - 1)
        def _():
            pltpu.sync_copy(src_hbm.at[pl.ds(r0 + 560, 64)], bnc.at[pl.ds(0, 64)])
            pltpu.sync_copy(bnc.at[pl.ds(0, 64)], tbl_sh.at[pl.ds(r0 + 560, 64)])

        @pl.when(sid == NS - 1)
        def _():
            pltpu.sync_copy(src_hbm.at[pl.ds(r0 + 560, CH)], bnc)
            pltpu.sync_copy(bnc, tbl_sh.at[pl.ds(r0 + 560, CH)])

        plsc.subcore_barrier()

    def phase(plane, ioff):
        out_pl = kq_out.at[plane]

        def idx_start(i):
            p = lax.rem(i, 2)
            off = (cbase + i) * 256
            pltpu.async_copy(epk_hbm.at[pl.ds(off, 256)], eidx2.at[p], semi)

        def idx_wait(i):
            p = lax.rem(i, 2)
            pltpu.make_async_copy(
                epk_hbm.at[pl.ds(0, 256)], eidx2.at[p], semi).wait()

        def g_start(i):
            p = lax.rem(i, 2)
            pltpu.async_copy(
                tbl_sh.at[eidx2.at[p].at[pl.ds(ioff, CH)]], rows2.at[p], semg)

        def g_wait(i):
            p = lax.rem(i, 2)
            pltpu.make_async_copy(
                tbl_sh.at[eidx2.at[p].at[pl.ds(ioff, CH)]], rows2.at[p],
                semg).wait()

        def wb_start(i):
            p = lax.rem(i, 2)
            off = base + i * CH
            pltpu.async_copy(rows2.at[p], out_pl.at[pl.ds(off, CH)], semw)

        def wb_wait(i):
            p = lax.rem(i, 2)
            pltpu.make_async_copy(
                rows2.at[p], out_pl.at[pl.ds(base, CH)], semw).wait()

        idx_start(0)
        idx_wait(0)
        g_start(0)
        idx_start(1)

        def body(i, carry):
            g_wait(i)
            wb_start(i)

            @pl.when(i + 1 < NCHUNK)
            def _():
                @pl.when(i >= 1)
                def _():
                    wb_wait(i - 1)

                idx_wait(i + 1)
                g_start(i + 1)

                @pl.when(i + 2 < NCHUNK)
                def _():
                    idx_start(i + 2)

            return carry

        lax.fori_loop(0, NCHUNK, body, 0)
        wb_wait(NCHUNK - 2)
        wb_wait(NCHUNK - 1)

    stage_tbl(k_hbm)
    phase(0, 0)
    plsc.subcore_barrier()
    stage_tbl(q_hbm)
    phase(1, 128)


def _score_body(ks_ref, qd_ref, attr_ref, we_ref, seg_ref, s_ref):
    eh = jnp.dot(attr_ref[...], we_ref[...], preferred_element_type=_f32)
    p = ks_ref[0] * qd_ref[0] * eh
    logit = jnp.dot(p, seg_ref[...], preferred_element_type=_f32)
    s_ref[...] = jnp.exp(jnp.clip(logit, -5.0, 5.0))


def _scores(kq, attr, w_e, seg):
    be = 8000
    return pl.pallas_call(
        _score_body,
        grid=(E // be,),
        in_specs=[
            pl.BlockSpec((1, be, D), lambda i: (0, i, 0)),
            pl.BlockSpec((1, be, D), lambda i: (1, i, 0)),
            pl.BlockSpec((be, DE), lambda i: (i, 0)),
            pl.BlockSpec((DE, D), lambda i: (0, 0)),
            pl.BlockSpec((D, DH), lambda i: (0, 0)),
        ],
        out_specs=pl.BlockSpec((be, DH), lambda i: (i, 0)),
        out_shape=jax.ShapeDtypeStruct((E, DH), _f32),
    )(kq, kq, attr, w_e, seg)


@functools.partial(
    pl.kernel,
    out_type=jax.ShapeDtypeStruct((NC, N, DW), _f32),
    mesh=_MESH,
    compiler_params=pltpu.CompilerParams(use_tc_tiling_on_sc=False),
    scratch_types=[
        pltpu.VMEM((2, 256), jnp.int32),
        pltpu.VMEM((2, HFA), jnp.int32),
        pltpu.VMEM((2, HFB), jnp.int32),
        pltpu.VMEM((2, CH, D), _f32),
        pltpu.VMEM((2, CH, DH), _f32),
        pltpu.VMEM((CH, DW), _f32),
        pltpu.VMEM_SHARED((N, DW), _f32),
        pltpu.SemaphoreType.DMA,
        pltpu.SemaphoreType.DMA,
        pltpu.SemaphoreType.DMA,
    ],
)
def _scatter_wv(v_hbm, epk_hbm, sc_hbm, tbl_out,
                eidx2, dscata, dscatb, vrows2, srows2, msg, tbl_sh,
                semi, semg, sems):
    cid = lax.axis_index("c")
    sid = lax.axis_index("s")
    wid = sid * NC + cid
    base = wid * EW
    r0 = sid * RPT

    zv = jnp.zeros((DH,), _f32)
    msg0 = msg

    def zb(r, c):
        for cc in range(DW // DH):
            msg[r, pl.ds(cc * DH, DH)] = zv
        return c

    lax.fori_loop(0, CH, zb, 0)

    def icp(j, c):
        pltpu.sync_copy(msg0, tbl_sh.at[pl.ds(r0 + j * CH, CH)])
        return c

    lax.fori_loop(0, 7, icp, 0)

    @pl.when(sid < NS - 1)
    def _():
        pltpu.sync_copy(msg0.at[pl.ds(0, 64)], tbl_sh.at[pl.ds(r0 + 560, 64)])

    @pl.when(sid == NS - 1)
    def _():
        pltpu.sync_copy(msg0, tbl_sh.at[pl.ds(r0 + 560, CH)])

    plsc.subcore_barrier()

    cbase = wid * NCHUNK

    def idx_start(i):
        p = lax.rem(i, 2)
        off = (cbase + i) * 256
        pltpu.async_copy(epk_hbm.at[pl.ds(off, 256)], eidx2.at[p], semi)
        pltpu.async_copy(sc_hbm.at[pl.ds(base + i * CH, CH)], srows2.at[p], semi)

    def idx_wait(i):
        p = lax.rem(i, 2)
        pltpu.make_async_copy(
            epk_hbm.at[pl.ds(0, 256)], eidx2.at[p], semi).wait()
        pltpu.make_async_copy(sc_hbm.at[pl.ds(base, CH)], srows2.at[p], semi).wait()

    def gv_start(i):
        p = lax.rem(i, 2)
        pltpu.async_copy(v_hbm.at[eidx2.at[p].at[pl.ds(0, CH)]], vrows2.at[p], semg)

    def gv_wait(i):
        p = lax.rem(i, 2)
        pltpu.make_async_copy(
            v_hbm.at[eidx2.at[p].at[pl.ds(0, CH)]], vrows2.at[p], semg).wait()

    def scat_start(i, hf):
        p = lax.rem(i, 2)
        if hf == 0:
            pltpu.async_copy(msg.at[pl.ds(0, HFA)],
                             tbl_sh.at[dscata.at[p]], sems, add=True)
        else:
            pltpu.async_copy(msg.at[pl.ds(HFA, HFB)],
                             tbl_sh.at[dscatb.at[p]], sems, add=True)

    def scat_wait(i, hf):
        p = lax.rem(i, 2)
        if hf == 0:
            pltpu.make_async_copy(msg.at[pl.ds(0, HFA)],
                                  tbl_sh.at[dscata.at[p]], sems).wait()
        else:
            pltpu.make_async_copy(msg.at[pl.ds(HFA, HFB)],
                                  tbl_sh.at[dscatb.at[p]], sems).wait()

    idx_start(0)
    idx_wait(0)
    gv_start(0)
    idx_start(1)

    def body(i, carry):
        p = lax.rem(i, 2)
        gv_wait(i)

        for b in range(HFA // DH):
            dscata[p, pl.ds(b * DH, DH)] = eidx2[p, pl.ds(128 + b * DH, DH)]
        for b in range(HFB // DH):
            dscatb[p, pl.ds(b * DH, DH)] = (
                eidx2[p, pl.ds(128 + HFA + b * DH, DH)])

        @pl.when(i + 1 < NCHUNK)
        def _():
            idx_wait(i + 1)
            gv_start(i + 1)

        @pl.when(i >= 1)
        def _():
            scat_wait(i - 1, 0)
            scat_wait(i - 1, 1)

        ihs = [jnp.full((DH,), h, jnp.int32) for h in range(H)]

        def ebody(e, c2):
            sv = srows2[p, e]
            for h in range(H):
                bh = jnp.take_along_axis(sv, ihs[h], axis=0)
                msg[e, pl.ds(h * DH, DH)] = (
                    vrows2[p, e, pl.ds(h * DH, DH)] * bh)
            msg[e, pl.ds(D, DH)] = sv
            return c2

        lax.fori_loop(0, HFA, ebody, 0, unroll=4)
        scat_start(i, 0)
        lax.fori_loop(HFA, CH, ebody, 0, unroll=4)

        @pl.when(i + 2 < NCHUNK)
        def _():
            idx_start(i + 2)

        scat_start(i, 1)
        return carry

    lax.fori_loop(0, NCHUNK, body, 0)
    scat_wait(NCHUNK - 1, 0)
    scat_wait(NCHUNK - 1, 1)

    plsc.subcore_barrier()

    def dcp(j, c):
        pltpu.sync_copy(tbl_sh.at[pl.ds(r0 + j * CH, CH)], msg0)
        pltpu.sync_copy(msg0, tbl_out.at[cid, pl.ds(r0 + j * CH, CH)])
        return c

    lax.fori_loop(0, 7, dcp, 0)

    @pl.when(sid < NS - 1)
    def _():
        pltpu.sync_copy(tbl_sh.at[pl.ds(r0 + 560, 64)], msg0.at[pl.ds(0, 64)])
        pltpu.sync_copy(msg0.at[pl.ds(0, 64)], tbl_out.at[cid, pl.ds(r0 + 560, 64)])

    @pl.when(sid == NS - 1)
    def _():
        pltpu.sync_copy(tbl_sh.at[pl.ds(r0 + 560, CH)], msg0)
        pltpu.sync_copy(msg0, tbl_out.at[cid, pl.ds(r0 + 560, CH)])


def _final_body(x_ref, t0_ref, t1_ref, pwv_ref, pz_ref,
                ln1g_ref, ln1b_ref, w1_ref, b1_ref, w2_ref, b2_ref,
                ln2g_ref, ln2b_ref, out_ref):
    t = t0_ref[0] + t1_ref[0]
    wv = jnp.dot(t, pwv_ref[...], preferred_element_type=_f32)
    zrep = jnp.dot(t, pz_ref[...], preferred_element_type=_f32)
    h = x_ref[...] + wv / (zrep + 1e-6)
    mu = jnp.mean(h, axis=-1, keepdims=True)
    var = jnp.mean((h - mu) ** 2, axis=-1, keepdims=True)
    h = (h - mu) / jnp.sqrt(var + 1e-5) * ln1g_ref[...] + ln1b_ref[...]
    f = jnp.dot(h, w1_ref[...], preferred_element_type=_f32) + b1_ref[...]
    f = jnp.maximum(f, 0.0)
    f = jnp.dot(f, w2_ref[...], preferred_element_type=_f32) + b2_ref[...]
    h = h + f
    mu = jnp.mean(h, axis=-1, keepdims=True)
    var = jnp.mean((h - mu) ** 2, axis=-1, keepdims=True)
    out_ref[...] = (h - mu) / jnp.sqrt(var + 1e-5) * ln2g_ref[...] + ln2b_ref[...]


def _final(x, tbl, pwv, pz, ln1g, ln1b, w1, b1, w2, b2, ln2g, ln2b):
    bn = 5000
    return pl.pallas_call(
        _final_body,
        grid=(N // bn,),
        in_specs=[
            pl.BlockSpec((bn, D), lambda i: (i, 0)),
            pl.BlockSpec((1, bn, DW), lambda i: (0, i, 0)),
            pl.BlockSpec((1, bn, DW), lambda i: (1, i, 0)),
            pl.BlockSpec((DW, D), lambda i: (0, 0)),
            pl.BlockSpec((DW, D), lambda i: (0, 0)),
            pl.BlockSpec((1, D), lambda i: (0, 0)),
            pl.BlockSpec((1, D), lambda i: (0, 0)),
            pl.BlockSpec((D, 2 * D), lambda i: (0, 0)),
            pl.BlockSpec((1, 2 * D), lambda i: (0, 0)),
            pl.BlockSpec((2 * D, D), lambda i: (0, 0)),
            pl.BlockSpec((1, D), lambda i: (0, 0)),
            pl.BlockSpec((1, D), lambda i: (0, 0)),
            pl.BlockSpec((1, D), lambda i: (0, 0)),
        ],
        out_specs=pl.BlockSpec((bn, D), lambda i: (i, 0)),
        out_shape=jax.ShapeDtypeStruct((N, D), _f32),
    )(x, tbl, tbl, pwv, pz, ln1g, ln1b, w1, b1, w2, b2, ln2g, ln2b)


def _seg_matrix():
    m = np.zeros((D, DH), np.float32)
    for h in range(H):
        m[h * DH:(h + 1) * DH, h] = 1.0 / np.sqrt(DH)
    return jnp.asarray(m)


def _pwv_matrix():
    m = np.zeros((DW, D), np.float32)
    m[:D, :] = np.eye(D, dtype=np.float32)
    return jnp.asarray(m)


def _pz_matrix():
    m = np.zeros((DW, D), np.float32)
    for h in range(H):
        m[D + h, h * DH:(h + 1) * DH] = 1.0
    return jnp.asarray(m)


def kernel(x, expander_edge_index, expander_edge_attr, W_Q, W_K, W_E, W_V,
           ln1_g, ln1_b, W1, b1, W2, b2, ln2_g, ln2_b):
    src = expander_edge_index[0]
    dst = expander_edge_index[1]
    epk = jnp.stack(
        [jnp.pad(src.reshape(-1, CH), ((0, 0), (0, 128 - CH))),
         jnp.pad(dst.reshape(-1, CH), ((0, 0), (0, 128 - CH)))],
        axis=1).reshape(-1)
    w_all = jnp.concatenate([W_Q, W_K, W_V], axis=1)

    q, k, v = _project(x, w_all)
    kq = _gather_kq(k, q, epk)
    scores = _scores(kq, expander_edge_attr, W_E, _seg_matrix())
    tbl = _scatter_wv(v, epk, scores)

    return _final(x, tbl, _pwv_matrix(), _pz_matrix(),
                  ln1_g.reshape(1, D), ln1_b.reshape(1, D),
                  W1, b1.reshape(1, 2 * D), W2, b2.reshape(1, D),
                  ln2_g.reshape(1, D), ln2_b.reshape(1, D))
